# baseline (speedup 1.0000x reference)
import jax
import jax.numpy as jnp
from jax import lax
from jax.experimental import pallas as pl
from jax.experimental.pallas import tpu as pltpu
from jax.experimental.pallas import tpu_sc as plsc

_BT = 128
_S = 50


def _splat(v):
    return jnp.full((16,), v, jnp.int32)


def _block(s, b, bt, w_hbm, out5_hbm, xcol_v, rows_v, trans_v,
           gsems, ssems, vrows, vdts, vdss):
    pltpu.make_async_copy(w_hbm.at[xcol_v.at[s]], rows_v.at[b],
                          gsems[b]).wait()

    @pl.when(s >= 2)
    def _():
        pltpu.make_async_copy(trans_v.at[b, :, :, pl.ds(0, _BT)],
                              out5_hbm.at[s, :, bt], ssems[b]).wait()

    trans_ref = trans_v.at[b]

    @plsc.parallel_loop(0, _BT, step=8, unroll=4)
    def bs_body(bs0):
        for j in range(8):
            bs = bs0 + j
            vbs = _splat(bs)
            vecs = [rows_v[b, bs, pl.ds(16 * dq, 16)] for dq in range(4)]
            for dq in range(4):
                plsc.store_scatter(trans_ref,
                                   [vdts[dq], vdss[dq], vbs], vecs[dq])

    @pl.when(s + 2 < _S)
    def _():
        pltpu.async_copy(w_hbm.at[xcol_v.at[s + 2]], rows_v.at[b], gsems[b])

    pltpu.async_copy(trans_v.at[b, :, :, pl.ds(0, _BT)],
                     out5_hbm.at[s, :, bt], ssems[b])


def _body(w_hbm, xr_hbm, out5_hbm, xcol_v, rows_v, trans_v, gsems, ssems):
    nc = plsc.get_sparse_core_info().num_cores
    wid = lax.axis_index("s") * nc + lax.axis_index("c")
    iota = lax.iota(jnp.int32, 16)
    vrows = [iota + 16 * g for g in range(8)]
    vdts = [(iota + 16 * g) >> 3 for g in range(4)]
    vdss = [(iota + 16 * g) & 7 for g in range(4)]

    for k in range(4):
        bt = wid * 4 + k
        pltpu.sync_copy(xr_hbm.at[:, pl.ds(bt * _BT, _BT)], xcol_v)
        for b in range(2):
            pltpu.async_copy(w_hbm.at[xcol_v.at[b]], rows_v.at[b], gsems[b])

        def outer(so, _, bt=bt):
            for b in range(2):
                _block(so * 2 + b, b, bt, w_hbm, out5_hbm, xcol_v,
                       rows_v, trans_v, gsems, ssems, vrows, vdts, vdss)
            return 0

        lax.fori_loop(0, _S // 2, outer, 0)
        for b in range(2):
            pltpu.make_async_copy(trans_v.at[b, :, :, pl.ds(0, _BT)],
                                  out5_hbm.at[_S - 2 + b, :, bt],
                                  ssems[b]).wait()


def kernel(x, W):
    Bt, S = x.shape
    V, D = W.shape
    assert S == _S and D == 64 and Bt % (_BT * 32) == 0
    nbt = Bt // _BT
    xr = x.T

    mesh = plsc.VectorSubcoreMesh(core_axis_name="c", subcore_axis_name="s")
    k = pl.kernel(
        _body,
        out_type=jax.ShapeDtypeStruct((S, 8, nbt, 8, _BT), jnp.float32),
        mesh=mesh,
        scratch_types=[
            pltpu.VMEM((_S, _BT), jnp.int32),
            pltpu.VMEM((2, _BT, D), jnp.float32),
            pltpu.VMEM((2, 8, 8, _BT + 1), jnp.float32),
            [pltpu.SemaphoreType.DMA] * 2,
            [pltpu.SemaphoreType.DMA] * 2,
        ],
        compiler_params=pltpu.CompilerParams(
            use_tc_tiling_on_sc=False,
            needs_layout_passes=False,
            disable_bounds_checks=True,
        ),
    )
    out5 = k(W, xr)
    return out5.transpose(2, 4, 0, 1, 3).reshape(Bt, S, D)


# speedup vs baseline: 1.0159x; 1.0159x over previous
import jax
import jax.numpy as jnp
from jax import lax
from jax.experimental import pallas as pl
from jax.experimental.pallas import tpu as pltpu
from jax.experimental.pallas import tpu_sc as plsc

_BT = 128
_S = 50


def _splat(v):
    return jnp.full((16,), v, jnp.int32)


def _block(s, b, bt, w_hbm, out5_hbm, xcol_v, rows_v, trans_v,
           gsems, ssems, vrows, vdts, vdss):
    pltpu.make_async_copy(w_hbm.at[xcol_v.at[s]], rows_v.at[b],
                          gsems[b]).wait()

    @pl.when(s >= 2)
    def _():
        pltpu.make_async_copy(trans_v.at[b, :, :, pl.ds(0, _BT)],
                              out5_hbm.at[s, :, bt], ssems[b]).wait()

    trans_ref = trans_v.at[b]

    @plsc.parallel_loop(0, _BT, step=8, unroll=2)
    def bs_body(bs0):
        for j in range(8):
            bs = bs0 + j
            vbs = _splat(bs)
            vecs = [rows_v[b, bs, pl.ds(16 * dq, 16)] for dq in range(4)]
            for dq in range(4):
                plsc.store_scatter(trans_ref,
                                   [vdts[dq], vdss[dq], vbs], vecs[dq])

    @pl.when(s + 2 < _S)
    def _():
        pltpu.async_copy(w_hbm.at[xcol_v.at[s + 2]], rows_v.at[b], gsems[b])

    pltpu.async_copy(trans_v.at[b, :, :, pl.ds(0, _BT)],
                     out5_hbm.at[s, :, bt], ssems[b])


def _body(w_hbm, xr_hbm, out5_hbm, xcol_v, rows_v, trans_v, gsems, ssems):
    nc = plsc.get_sparse_core_info().num_cores
    wid = lax.axis_index("s") * nc + lax.axis_index("c")
    iota = lax.iota(jnp.int32, 16)
    vrows = [iota + 16 * g for g in range(8)]
    vdts = [(iota + 16 * g) >> 3 for g in range(4)]
    vdss = [(iota + 16 * g) & 7 for g in range(4)]

    for k in range(4):
        bt = wid * 4 + k
        pltpu.sync_copy(xr_hbm.at[:, pl.ds(bt * _BT, _BT)], xcol_v)
        for b in range(2):
            pltpu.async_copy(w_hbm.at[xcol_v.at[b]], rows_v.at[b], gsems[b])

        def outer(so, _, bt=bt):
            for b in range(2):
                _block(so * 2 + b, b, bt, w_hbm, out5_hbm, xcol_v,
                       rows_v, trans_v, gsems, ssems, vrows, vdts, vdss)
            return 0

        lax.fori_loop(0, _S // 2, outer, 0)
        for b in range(2):
            pltpu.make_async_copy(trans_v.at[b, :, :, pl.ds(0, _BT)],
                                  out5_hbm.at[_S - 2 + b, :, bt],
                                  ssems[b]).wait()


def kernel(x, W):
    Bt, S = x.shape
    V, D = W.shape
    assert S == _S and D == 64 and Bt % (_BT * 32) == 0
    nbt = Bt // _BT
    xr = x.T

    mesh = plsc.VectorSubcoreMesh(core_axis_name="c", subcore_axis_name="s")
    k = pl.kernel(
        _body,
        out_type=jax.ShapeDtypeStruct((S, 8, nbt, 8, _BT), jnp.float32),
        mesh=mesh,
        scratch_types=[
            pltpu.VMEM((_S, _BT), jnp.int32),
            pltpu.VMEM((2, _BT, D), jnp.float32),
            pltpu.VMEM((2, 8, 8, _BT + 1), jnp.float32),
            [pltpu.SemaphoreType.DMA] * 2,
            [pltpu.SemaphoreType.DMA] * 2,
        ],
        compiler_params=pltpu.CompilerParams(
            use_tc_tiling_on_sc=False,
            needs_layout_passes=False,
            disable_bounds_checks=True,
        ),
    )
    out5 = k(W, xr)
    return out5.transpose(2, 4, 0, 1, 3).reshape(Bt, S, D)
